# Initial kernel scaffold; baseline (speedup 1.0000x reference)
import jax
import jax.numpy as jnp
from jax.experimental import pallas as pl


def kernel(x, edge_index1, etypes1, edge_index2, etypes2, W1, Ws1, b1, gamma, beta, W2, Ws2, b2):
    raise NotImplementedError("write your pallas kernel here")


# speedup vs baseline: 19.2153x; 19.2153x over previous
import functools

import jax
import jax.numpy as jnp
from jax import lax
from jax.experimental import pallas as pl
from jax.experimental.pallas import tpu as pltpu
from jax.experimental.pallas import tpu_sc as plsc

N = 10000
E = 320000
R = 4
D_IN = 128
D_H = 128
D_OUT = 40
D_OUT_P = 128
BN_EPS = 1e-5

NC = 2
NS = 16
NW = NC * NS
EPT = E // NW
ROWS_PT = 624
ROWS_TAIL = N - NS * ROWS_PT

BM = 400
C = 80


def _xform1_body(x_ref, w_ref, ws_ref, b_ref, t_ref, s_ref):
    xb = x_ref[...]
    for r in range(R):
        t_ref[r] = jnp.dot(xb, w_ref[r], preferred_element_type=jnp.float32)
    s_ref[0] = jnp.dot(xb, ws_ref[...], preferred_element_type=jnp.float32) + b_ref[...]
    s_ref[1] = jnp.zeros((BM, D_H), jnp.float32)


@jax.jit
def _xform1(x, W1, Ws1, b1):
    return pl.pallas_call(
        _xform1_body,
        grid=(N // BM,),
        in_specs=[
            pl.BlockSpec((BM, D_IN), lambda i: (i, 0)),
            pl.BlockSpec((R, D_IN, D_H), lambda i: (0, 0, 0)),
            pl.BlockSpec((D_IN, D_H), lambda i: (0, 0)),
            pl.BlockSpec((1, D_H), lambda i: (0, 0)),
        ],
        out_specs=[
            pl.BlockSpec((R, BM, D_H), lambda i: (0, i, 0)),
            pl.BlockSpec((NC, BM, D_H), lambda i: (0, i, 0)),
        ],
        out_shape=[
            jax.ShapeDtypeStruct((R, N, D_H), jnp.float32),
            jax.ShapeDtypeStruct((NC, N, D_H), jnp.float32),
        ],
    )(x, W1, Ws1, b1)


def _xform2_body(acc_ref, g_ref, be_ref, w_ref, ws_ref, b_ref, t_ref, s_ref):
    inv = 1.0 / jnp.sqrt(1.0 + BN_EPS)
    h = (acc_ref[0] + acc_ref[1]) * inv
    h = g_ref[...] * h + be_ref[...]
    h = jnp.maximum(h, 0.0)
    for r in range(R):
        t_ref[r] = jnp.dot(h, w_ref[r], preferred_element_type=jnp.float32)
    s_ref[0] = jnp.dot(h, ws_ref[...], preferred_element_type=jnp.float32) + b_ref[...]
    s_ref[1] = jnp.zeros((BM, D_OUT_P), jnp.float32)


@jax.jit
def _xform2(acc, gamma, beta, W2p, Ws2p, b2p):
    return pl.pallas_call(
        _xform2_body,
        grid=(N // BM,),
        in_specs=[
            pl.BlockSpec((NC, BM, D_H), lambda i: (0, i, 0)),
            pl.BlockSpec((1, D_H), lambda i: (0, 0)),
            pl.BlockSpec((1, D_H), lambda i: (0, 0)),
            pl.BlockSpec((R, D_H, D_OUT_P), lambda i: (0, 0, 0)),
            pl.BlockSpec((D_H, D_OUT_P), lambda i: (0, 0)),
            pl.BlockSpec((1, D_OUT_P), lambda i: (0, 0)),
        ],
        out_specs=[
            pl.BlockSpec((R, BM, D_OUT_P), lambda i: (0, i, 0)),
            pl.BlockSpec((NC, BM, D_OUT_P), lambda i: (0, i, 0)),
        ],
        out_shape=[
            jax.ShapeDtypeStruct((R, N, D_OUT_P), jnp.float32),
            jax.ShapeDtypeStruct((NC, N, D_OUT_P), jnp.float32),
        ],
    )(acc, gamma, beta, W2p, Ws2p, b2p)


def _finish_body(acc_ref, o_ref):
    s = (acc_ref[0] + acc_ref[1])[:, :D_OUT]
    m = jnp.max(s, axis=1, keepdims=True)
    e = jnp.exp(s - m)
    lse = m + jnp.log(jnp.sum(e, axis=1, keepdims=True))
    o_ref[...] = s - lse


@jax.jit
def _finish(acc):
    return pl.pallas_call(
        _finish_body,
        grid=(N // BM,),
        in_specs=[pl.BlockSpec((NC, BM, D_OUT_P), lambda i: (0, i, 0))],
        out_specs=pl.BlockSpec((BM, D_OUT), lambda i: (i, 0)),
        out_shape=jax.ShapeDtypeStruct((N, D_OUT), jnp.float32),
    )(acc)


def _make_agg(D):
    nch = EPT // C
    mesh = plsc.VectorSubcoreMesh(core_axis_name="c", subcore_axis_name="s")

    @functools.partial(
        pl.kernel,
        out_type=jax.ShapeDtypeStruct((NC, N, D), jnp.float32),
        mesh=mesh,
        scratch_types=[
            pltpu.VMEM((EPT,), jnp.int32),
            pltpu.VMEM((EPT,), jnp.int32),
            pltpu.VMEM((C,), jnp.int32),
            pltpu.VMEM((C, D), jnp.float32),
            pltpu.VMEM_SHARED((N, D), jnp.float32),
            pltpu.SemaphoreType.DMA,
        ],
    )
    def agg(src_hbm, dst_hbm, et_hbm, table_hbm, init_hbm, out_hbm,
            gidx_v, et_v, dst_v, rows_v, acc_s, sem):
        c = lax.axis_index("c")
        s = lax.axis_index("s")
        wid = s * NC + c
        ebase = wid * EPT

        pltpu.sync_copy(init_hbm.at[c, pl.ds(s * ROWS_PT, ROWS_PT)],
                        acc_s.at[pl.ds(s * ROWS_PT, ROWS_PT)])

        @pl.when(s == NS - 1)
        def _():
            pltpu.sync_copy(init_hbm.at[c, pl.ds(NS * ROWS_PT, ROWS_TAIL)],
                            acc_s.at[pl.ds(NS * ROWS_PT, ROWS_TAIL)])

        pltpu.sync_copy(src_hbm.at[pl.ds(ebase, EPT)], gidx_v)
        pltpu.sync_copy(et_hbm.at[pl.ds(ebase, EPT)], et_v)

        def cvt(i, _):
            sl = pl.ds(i * 16, 16)
            gidx_v[sl] = et_v[sl] * N + gidx_v[sl]
            return 0
        lax.fori_loop(0, EPT // 16, cvt, 0, unroll=8)

        plsc.subcore_barrier()

        def chunk(k, _):
            pltpu.sync_copy(dst_hbm.at[pl.ds(ebase + k * C, C)], dst_v)
            pltpu.async_copy(table_hbm.at[gidx_v.at[pl.ds(k * C, C)]],
                             rows_v, sem).wait()
            pltpu.sync_copy(rows_v, acc_s.at[dst_v], add=True)
            return 0
        lax.fori_loop(0, nch, chunk, 0)

        plsc.subcore_barrier()
        pltpu.sync_copy(acc_s.at[pl.ds(s * ROWS_PT, ROWS_PT)],
                        out_hbm.at[c, pl.ds(s * ROWS_PT, ROWS_PT)])

        @pl.when(s == NS - 1)
        def _():
            pltpu.sync_copy(acc_s.at[pl.ds(NS * ROWS_PT, ROWS_TAIL)],
                            out_hbm.at[c, pl.ds(NS * ROWS_PT, ROWS_TAIL)])

    return agg


_agg_h = jax.jit(_make_agg(D_H))
_agg_o = jax.jit(_make_agg(D_OUT_P))


def kernel(x, edge_index1, etypes1, edge_index2, etypes2,
           W1, Ws1, b1, gamma, beta, W2, Ws2, b2):
    T1, init1 = _xform1(x, W1, Ws1, b1.reshape(1, D_H))
    acc1 = _agg_h(edge_index1[0], edge_index1[1], etypes1,
                  T1.reshape(R * N, D_H), init1)

    W2p = jnp.pad(W2, ((0, 0), (0, 0), (0, D_OUT_P - D_OUT)))
    Ws2p = jnp.pad(Ws2, ((0, 0), (0, D_OUT_P - D_OUT)))
    b2p = jnp.pad(b2, ((0, D_OUT_P - D_OUT),)).reshape(1, D_OUT_P)
    T2, init2 = _xform2(acc1, gamma.reshape(1, D_H), beta.reshape(1, D_H),
                        W2p, Ws2p, b2p)
    acc2 = _agg_o(edge_index2[0], edge_index2[1], etypes2,
                  T2.reshape(R * N, D_OUT_P), init2)
    return _finish(acc2)


# speedup vs baseline: 34.6963x; 1.8057x over previous
import functools

import jax
import jax.numpy as jnp
from jax import lax
from jax.experimental import pallas as pl
from jax.experimental.pallas import tpu as pltpu
from jax.experimental.pallas import tpu_sc as plsc

N = 10000
E = 320000
R = 4
D_IN = 128
D_H = 128
D_OUT = 40
D_OUT_P = 128
BN_EPS = 1e-5

NC = 2
NS = 16
NW = NC * NS
EPT = E // NW
ROWS_PT = 624
ROWS_TAIL = N - NS * ROWS_PT

BM = 400
C = 80


def _xform1_body(x_ref, w_ref, ws_ref, b_ref, t_ref, s_ref):
    xb = x_ref[...]
    for r in range(R):
        t_ref[r] = jnp.dot(xb, w_ref[r], preferred_element_type=jnp.float32)
    s_ref[0] = jnp.dot(xb, ws_ref[...], preferred_element_type=jnp.float32) + b_ref[...]
    s_ref[1] = jnp.zeros((BM, D_H), jnp.float32)


@jax.jit
def _xform1(x, W1, Ws1, b1):
    return pl.pallas_call(
        _xform1_body,
        grid=(N // BM,),
        in_specs=[
            pl.BlockSpec((BM, D_IN), lambda i: (i, 0)),
            pl.BlockSpec((R, D_IN, D_H), lambda i: (0, 0, 0)),
            pl.BlockSpec((D_IN, D_H), lambda i: (0, 0)),
            pl.BlockSpec((1, D_H), lambda i: (0, 0)),
        ],
        out_specs=[
            pl.BlockSpec((R, BM, D_H), lambda i: (0, i, 0)),
            pl.BlockSpec((NC, BM, D_H), lambda i: (0, i, 0)),
        ],
        out_shape=[
            jax.ShapeDtypeStruct((R, N, D_H), jnp.float32),
            jax.ShapeDtypeStruct((NC, N, D_H), jnp.float32),
        ],
    )(x, W1, Ws1, b1)


def _xform2_body(acc_ref, g_ref, be_ref, w_ref, ws_ref, b_ref, t_ref, s_ref):
    inv = 1.0 / jnp.sqrt(1.0 + BN_EPS)
    h = (acc_ref[0] + acc_ref[1]) * inv
    h = g_ref[...] * h + be_ref[...]
    h = jnp.maximum(h, 0.0)
    for r in range(R):
        t_ref[r] = jnp.dot(h, w_ref[r], preferred_element_type=jnp.float32)
    s_ref[0] = jnp.dot(h, ws_ref[...], preferred_element_type=jnp.float32) + b_ref[...]
    s_ref[1] = jnp.zeros((BM, D_OUT_P), jnp.float32)


@jax.jit
def _xform2(acc, gamma, beta, W2p, Ws2p, b2p):
    return pl.pallas_call(
        _xform2_body,
        grid=(N // BM,),
        in_specs=[
            pl.BlockSpec((NC, BM, D_H), lambda i: (0, i, 0)),
            pl.BlockSpec((1, D_H), lambda i: (0, 0)),
            pl.BlockSpec((1, D_H), lambda i: (0, 0)),
            pl.BlockSpec((R, D_H, D_OUT_P), lambda i: (0, 0, 0)),
            pl.BlockSpec((D_H, D_OUT_P), lambda i: (0, 0)),
            pl.BlockSpec((1, D_OUT_P), lambda i: (0, 0)),
        ],
        out_specs=[
            pl.BlockSpec((R, BM, D_OUT_P), lambda i: (0, i, 0)),
            pl.BlockSpec((NC, BM, D_OUT_P), lambda i: (0, i, 0)),
        ],
        out_shape=[
            jax.ShapeDtypeStruct((R, N, D_OUT_P), jnp.float32),
            jax.ShapeDtypeStruct((NC, N, D_OUT_P), jnp.float32),
        ],
    )(acc, gamma, beta, W2p, Ws2p, b2p)


def _finish_body(acc_ref, o_ref):
    s = (acc_ref[0] + acc_ref[1])[:, :D_OUT]
    m = jnp.max(s, axis=1, keepdims=True)
    e = jnp.exp(s - m)
    lse = m + jnp.log(jnp.sum(e, axis=1, keepdims=True))
    o_ref[...] = s - lse


@jax.jit
def _finish(acc):
    return pl.pallas_call(
        _finish_body,
        grid=(N // BM,),
        in_specs=[pl.BlockSpec((NC, BM, D_OUT_P), lambda i: (0, i, 0))],
        out_specs=pl.BlockSpec((BM, D_OUT), lambda i: (i, 0)),
        out_shape=jax.ShapeDtypeStruct((N, D_OUT), jnp.float32),
    )(acc)


def _make_agg(D):
    nch = EPT // C
    mesh = plsc.VectorSubcoreMesh(core_axis_name="c", subcore_axis_name="s")

    @functools.partial(
        pl.kernel,
        out_type=jax.ShapeDtypeStruct((NC, N, D), jnp.float32),
        mesh=mesh,
        scratch_types=[
            pltpu.VMEM((EPT,), jnp.int32),
            pltpu.VMEM((EPT,), jnp.int32),
            pltpu.VMEM((C,), jnp.int32),
            pltpu.VMEM((C,), jnp.int32),
            pltpu.VMEM((C, D), jnp.float32),
            pltpu.VMEM((C, D), jnp.float32),
            pltpu.VMEM_SHARED((N, D), jnp.float32),
            pltpu.SemaphoreType.DMA,
            pltpu.SemaphoreType.DMA,
            pltpu.SemaphoreType.DMA,
            pltpu.SemaphoreType.DMA,
        ],
    )
    def agg(src_hbm, dst_hbm, et_hbm, table_hbm, init_hbm, out_hbm,
            gidx_v, et_v, dst0_v, dst1_v, rows0_v, rows1_v, acc_s,
            sem0, sem1, sem2, sem3):
        c = lax.axis_index("c")
        s = lax.axis_index("s")
        wid = s * NC + c
        ebase = wid * EPT

        pltpu.sync_copy(init_hbm.at[c, pl.ds(s * ROWS_PT, ROWS_PT)],
                        acc_s.at[pl.ds(s * ROWS_PT, ROWS_PT)])

        @pl.when(s == NS - 1)
        def _():
            pltpu.sync_copy(init_hbm.at[c, pl.ds(NS * ROWS_PT, ROWS_TAIL)],
                            acc_s.at[pl.ds(NS * ROWS_PT, ROWS_TAIL)])

        pltpu.sync_copy(src_hbm.at[pl.ds(ebase, EPT)], gidx_v)
        pltpu.sync_copy(et_hbm.at[pl.ds(ebase, EPT)], et_v)

        def cvt(i, _):
            sl = pl.ds(i * 16, 16)
            gidx_v[sl] = et_v[sl] * N + gidx_v[sl]
            return 0
        lax.fori_loop(0, EPT // 16, cvt, 0, unroll=8)

        plsc.subcore_barrier()

        def gather(k, buf, sem):
            return pltpu.make_async_copy(
                table_hbm.at[gidx_v.at[pl.ds(k * C, C)]], buf, sem)

        def dstcp(k, buf, sem):
            return pltpu.make_async_copy(
                dst_hbm.at[pl.ds(ebase + k * C, C)], buf, sem)

        gather(0, rows0_v, sem0).start()
        dstcp(0, dst0_v, sem2).start()

        def chunkpair(kk, _):
            k0 = 2 * kk
            gather(k0 + 1, rows1_v, sem1).start()
            dstcp(k0 + 1, dst1_v, sem3).start()
            gather(k0, rows0_v, sem0).wait()
            dstcp(k0, dst0_v, sem2).wait()
            pltpu.sync_copy(rows0_v, acc_s.at[dst0_v], add=True)
            gather(k0 + 2, rows0_v, sem0).start()
            dstcp(k0 + 2, dst0_v, sem2).start()
            gather(k0 + 1, rows1_v, sem1).wait()
            dstcp(k0 + 1, dst1_v, sem3).wait()
            pltpu.sync_copy(rows1_v, acc_s.at[dst1_v], add=True)
            return 0
        lax.fori_loop(0, (nch - 1) // 2, chunkpair, 0)

        gather(nch - 1, rows0_v, sem0).wait()
        dstcp(nch - 1, dst0_v, sem2).wait()
        pltpu.sync_copy(rows0_v, acc_s.at[dst0_v], add=True)

        plsc.subcore_barrier()
        pltpu.sync_copy(acc_s.at[pl.ds(s * ROWS_PT, ROWS_PT)],
                        out_hbm.at[c, pl.ds(s * ROWS_PT, ROWS_PT)])

        @pl.when(s == NS - 1)
        def _():
            pltpu.sync_copy(acc_s.at[pl.ds(NS * ROWS_PT, ROWS_TAIL)],
                            out_hbm.at[c, pl.ds(NS * ROWS_PT, ROWS_TAIL)])

    return agg


_agg_h = jax.jit(_make_agg(D_H))
_agg_o = jax.jit(_make_agg(D_OUT_P))


def kernel(x, edge_index1, etypes1, edge_index2, etypes2,
           W1, Ws1, b1, gamma, beta, W2, Ws2, b2):
    T1, init1 = _xform1(x, W1, Ws1, b1.reshape(1, D_H))
    acc1 = _agg_h(edge_index1[0], edge_index1[1],
                  etypes1, T1.reshape(R * N, D_H), init1)

    W2p = jnp.pad(W2, ((0, 0), (0, 0), (0, D_OUT_P - D_OUT)))
    Ws2p = jnp.pad(Ws2, ((0, 0), (0, D_OUT_P - D_OUT)))
    b2p = jnp.pad(b2, ((0, D_OUT_P - D_OUT),)).reshape(1, D_OUT_P)
    T2, init2 = _xform2(acc1, gamma.reshape(1, D_H), beta.reshape(1, D_H),
                        W2p, Ws2p, b2p)
    acc2 = _agg_o(edge_index2[0], edge_index2[1],
                  etypes2, T2.reshape(R * N, D_OUT_P), init2)
    return _finish(acc2)


# speedup vs baseline: 38.2045x; 1.1011x over previous
import functools

import jax
import jax.numpy as jnp
from jax import lax
from jax.experimental import pallas as pl
from jax.experimental.pallas import tpu as pltpu
from jax.experimental.pallas import tpu_sc as plsc

N = 10000
E = 320000
R = 4
D_IN = 128
D_H = 128
D_OUT = 40
D_OUT_P = 128
BN_EPS = 1e-5

NC = 2
NS = 16
NW = NC * NS
EPT = E // NW
ROWS_PT = 624
ROWS_TAIL = N - NS * ROWS_PT

BM = 400
C = 80


def _xform1_body(x_ref, w_ref, ws_ref, b_ref, t_ref, s_ref):
    xb = x_ref[...]
    for r in range(R):
        t_ref[r] = jnp.dot(xb, w_ref[r], preferred_element_type=jnp.float32)
    s_ref[0] = jnp.dot(xb, ws_ref[...], preferred_element_type=jnp.float32) + b_ref[...]
    s_ref[1] = jnp.zeros((BM, D_H), jnp.float32)


@jax.jit
def _xform1(x, W1, Ws1, b1):
    return pl.pallas_call(
        _xform1_body,
        grid=(N // BM,),
        in_specs=[
            pl.BlockSpec((BM, D_IN), lambda i: (i, 0)),
            pl.BlockSpec((R, D_IN, D_H), lambda i: (0, 0, 0)),
            pl.BlockSpec((D_IN, D_H), lambda i: (0, 0)),
            pl.BlockSpec((1, D_H), lambda i: (0, 0)),
        ],
        out_specs=[
            pl.BlockSpec((R, BM, D_H), lambda i: (0, i, 0)),
            pl.BlockSpec((NC, BM, D_H), lambda i: (0, i, 0)),
        ],
        out_shape=[
            jax.ShapeDtypeStruct((R, N, D_H), jnp.float32),
            jax.ShapeDtypeStruct((NC, N, D_H), jnp.float32),
        ],
    )(x, W1, Ws1, b1)


def _xform2_body(acc_ref, g_ref, be_ref, w_ref, ws_ref, b_ref, t_ref, s_ref):
    inv = 1.0 / jnp.sqrt(1.0 + BN_EPS)
    h = (acc_ref[0] + acc_ref[1]) * inv
    h = g_ref[...] * h + be_ref[...]
    h = jnp.maximum(h, 0.0)
    for r in range(R):
        t_ref[r] = jnp.dot(h, w_ref[r], preferred_element_type=jnp.float32)
    s_ref[0] = jnp.dot(h, ws_ref[...], preferred_element_type=jnp.float32) + b_ref[...]
    s_ref[1] = jnp.zeros((BM, D_OUT_P), jnp.float32)


@jax.jit
def _xform2(acc, gamma, beta, W2p, Ws2p, b2p):
    return pl.pallas_call(
        _xform2_body,
        grid=(N // BM,),
        in_specs=[
            pl.BlockSpec((NC, BM, D_H), lambda i: (0, i, 0)),
            pl.BlockSpec((1, D_H), lambda i: (0, 0)),
            pl.BlockSpec((1, D_H), lambda i: (0, 0)),
            pl.BlockSpec((R, D_H, D_OUT_P), lambda i: (0, 0, 0)),
            pl.BlockSpec((D_H, D_OUT_P), lambda i: (0, 0)),
            pl.BlockSpec((1, D_OUT_P), lambda i: (0, 0)),
        ],
        out_specs=[
            pl.BlockSpec((R, BM, D_OUT_P), lambda i: (0, i, 0)),
            pl.BlockSpec((NC, BM, D_OUT_P), lambda i: (0, i, 0)),
        ],
        out_shape=[
            jax.ShapeDtypeStruct((R, N, D_OUT_P), jnp.float32),
            jax.ShapeDtypeStruct((NC, N, D_OUT_P), jnp.float32),
        ],
    )(acc, gamma, beta, W2p, Ws2p, b2p)


def _finish_body(acc_ref, o_ref):
    s = (acc_ref[0] + acc_ref[1])[:, :D_OUT]
    m = jnp.max(s, axis=1, keepdims=True)
    e = jnp.exp(s - m)
    lse = m + jnp.log(jnp.sum(e, axis=1, keepdims=True))
    o_ref[...] = s - lse


@jax.jit
def _finish(acc):
    return pl.pallas_call(
        _finish_body,
        grid=(N // BM,),
        in_specs=[pl.BlockSpec((NC, BM, D_OUT_P), lambda i: (0, i, 0))],
        out_specs=pl.BlockSpec((BM, D_OUT), lambda i: (i, 0)),
        out_shape=jax.ShapeDtypeStruct((N, D_OUT), jnp.float32),
    )(acc)


def _make_agg(D):
    nch = EPT // C
    mesh = plsc.VectorSubcoreMesh(core_axis_name="c", subcore_axis_name="s")

    ETC = 2000
    NB = 3

    @functools.partial(
        pl.kernel,
        out_type=jax.ShapeDtypeStruct((NC, N, D), jnp.float32),
        mesh=mesh,
        scratch_types=[
            pltpu.VMEM((EPT,), jnp.int32),
            pltpu.VMEM((ETC,), jnp.int32),
            [pltpu.VMEM((C,), jnp.int32) for _ in range(NB)],
            [pltpu.VMEM((C, D), jnp.float32) for _ in range(NB)],
            pltpu.VMEM_SHARED((N, D), jnp.float32),
            [pltpu.SemaphoreType.DMA for _ in range(3 * NB)],
        ],
    )
    def agg(src_hbm, dst_hbm, et_hbm, table_hbm, init_hbm, out_hbm,
            gidx_v, et_v, dst_bufs, row_bufs, acc_s, sems):
        c = lax.axis_index("c")
        s = lax.axis_index("s")
        wid = s * NC + c
        ebase = wid * EPT
        sg = sems[0:NB]
        sd = sems[NB:2 * NB]
        ss = sems[2 * NB:]

        pltpu.sync_copy(init_hbm.at[c, pl.ds(s * ROWS_PT, ROWS_PT)],
                        acc_s.at[pl.ds(s * ROWS_PT, ROWS_PT)])

        @pl.when(s == NS - 1)
        def _():
            pltpu.sync_copy(init_hbm.at[c, pl.ds(NS * ROWS_PT, ROWS_TAIL)],
                            acc_s.at[pl.ds(NS * ROWS_PT, ROWS_TAIL)])

        pltpu.sync_copy(src_hbm.at[pl.ds(ebase, EPT)], gidx_v)

        def stage(j, _):
            pltpu.sync_copy(et_hbm.at[pl.ds(ebase + j * ETC, ETC)], et_v)

            def cvt(i, _):
                sl = pl.ds(j * ETC + i * 16, 16)
                gidx_v[sl] = et_v[pl.ds(i * 16, 16)] * N + gidx_v[sl]
                return 0
            lax.fori_loop(0, ETC // 16, cvt, 0, unroll=8)
            return 0
        lax.fori_loop(0, EPT // ETC, stage, 0)

        plsc.subcore_barrier()

        def gather(k, b):
            return pltpu.make_async_copy(
                table_hbm.at[gidx_v.at[pl.ds(k * C, C)]], row_bufs[b], sg[b])

        def dstcp(k, b):
            return pltpu.make_async_copy(
                dst_hbm.at[pl.ds(ebase + k * C, C)], dst_bufs[b], sd[b])

        def scat_start(b):
            pltpu.async_copy(row_bufs[b], acc_s.at[dst_bufs[b]], ss[b],
                             add=True)

        def scat_wait(b):
            pltpu.make_async_copy(row_bufs[b], acc_s.at[dst_bufs[b]],
                                  ss[b]).wait()

        def slot(k, b, bn, first, last):
            if not first:
                scat_wait(bn)
            if not last:
                gather(k + 1, bn).start()
                dstcp(k + 1, bn).start()
            gather(k, b).wait()
            dstcp(k, b).wait()
            scat_start(b)

        gather(0, 0).start()
        dstcp(0, 0).start()
        slot(0, 0, 1, True, False)
        slot(1, 1, 2, True, False)

        def body3(kk, _):
            k = 3 * kk + 2
            slot(k, 2, 0, False, False)
            slot(k + 1, 0, 1, False, False)
            slot(k + 2, 1, 2, False, False)
            return 0
        lax.fori_loop(0, (nch - 5) // 3, body3, 0)

        slot(nch - 3, 2, 0, False, False)
        slot(nch - 2, 0, 1, False, False)
        slot(nch - 1, 1, 2, False, True)
        scat_wait(0)
        scat_wait(1)

        plsc.subcore_barrier()
        pltpu.sync_copy(acc_s.at[pl.ds(s * ROWS_PT, ROWS_PT)],
                        out_hbm.at[c, pl.ds(s * ROWS_PT, ROWS_PT)])

        @pl.when(s == NS - 1)
        def _():
            pltpu.sync_copy(acc_s.at[pl.ds(NS * ROWS_PT, ROWS_TAIL)],
                            out_hbm.at[c, pl.ds(NS * ROWS_PT, ROWS_TAIL)])

    return agg


_agg_h = jax.jit(_make_agg(D_H))
_agg_o = jax.jit(_make_agg(D_OUT_P))


def kernel(x, edge_index1, etypes1, edge_index2, etypes2,
           W1, Ws1, b1, gamma, beta, W2, Ws2, b2):
    T1, init1 = _xform1(x, W1, Ws1, b1.reshape(1, D_H))
    acc1 = _agg_h(edge_index1[0], edge_index1[1],
                  etypes1, T1.reshape(R * N, D_H), init1)

    W2p = jnp.pad(W2, ((0, 0), (0, 0), (0, D_OUT_P - D_OUT)))
    Ws2p = jnp.pad(Ws2, ((0, 0), (0, D_OUT_P - D_OUT)))
    b2p = jnp.pad(b2, ((0, D_OUT_P - D_OUT),)).reshape(1, D_OUT_P)
    T2, init2 = _xform2(acc1, gamma.reshape(1, D_H), beta.reshape(1, D_H),
                        W2p, Ws2p, b2p)
    acc2 = _agg_o(edge_index2[0], edge_index2[1],
                  etypes2, T2.reshape(R * N, D_OUT_P), init2)
    return _finish(acc2)


# speedup vs baseline: 40.4666x; 1.0592x over previous
import functools

import jax
import jax.numpy as jnp
from jax import lax
from jax.experimental import pallas as pl
from jax.experimental.pallas import tpu as pltpu
from jax.experimental.pallas import tpu_sc as plsc

N = 10000
E = 320000
R = 4
D_IN = 128
D_H = 128
D_OUT = 40
D_OUT_P = 128
BN_EPS = 1e-5

NC = 2
NS = 16
NW = NC * NS
EPT = E // NW
ROWS_PT = 624
ROWS_TAIL = N - NS * ROWS_PT

BM = 400
C = 80


def _xform1_body(x_ref, w_ref, ws_ref, b_ref, t_ref, s_ref):
    xb = x_ref[...].astype(jnp.bfloat16)
    for r in range(R):
        t_ref[r] = jnp.dot(xb, w_ref[r].astype(jnp.bfloat16),
                           preferred_element_type=jnp.float32)
    s_ref[0] = jnp.dot(xb, ws_ref[...].astype(jnp.bfloat16),
                       preferred_element_type=jnp.float32) + b_ref[...]
    s_ref[1] = jnp.zeros((BM, D_H), jnp.float32)


@jax.jit
def _xform1(x, W1, Ws1, b1):
    return pl.pallas_call(
        _xform1_body,
        grid=(N // BM,),
        in_specs=[
            pl.BlockSpec((BM, D_IN), lambda i: (i, 0)),
            pl.BlockSpec((R, D_IN, D_H), lambda i: (0, 0, 0)),
            pl.BlockSpec((D_IN, D_H), lambda i: (0, 0)),
            pl.BlockSpec((1, D_H), lambda i: (0, 0)),
        ],
        out_specs=[
            pl.BlockSpec((R, BM, D_H), lambda i: (0, i, 0)),
            pl.BlockSpec((NC, BM, D_H), lambda i: (0, i, 0)),
        ],
        out_shape=[
            jax.ShapeDtypeStruct((R, N, D_H), jnp.float32),
            jax.ShapeDtypeStruct((NC, N, D_H), jnp.float32),
        ],
    )(x, W1, Ws1, b1)


def _xform2_body(acc_ref, g_ref, be_ref, w_ref, ws_ref, b_ref, t_ref, s_ref):
    inv = 1.0 / jnp.sqrt(1.0 + BN_EPS)
    h = (acc_ref[0] + acc_ref[1]) * inv
    h = g_ref[...] * h + be_ref[...]
    h = jnp.maximum(h, 0.0)
    hb = h.astype(jnp.bfloat16)
    for r in range(R):
        t_ref[r] = jnp.dot(hb, w_ref[r].astype(jnp.bfloat16),
                           preferred_element_type=jnp.float32)
    s_ref[0] = jnp.dot(hb, ws_ref[...].astype(jnp.bfloat16),
                       preferred_element_type=jnp.float32) + b_ref[...]
    s_ref[1] = jnp.zeros((BM, D_OUT_P), jnp.float32)


@jax.jit
def _xform2(acc, gamma, beta, W2p, Ws2p, b2p):
    return pl.pallas_call(
        _xform2_body,
        grid=(N // BM,),
        in_specs=[
            pl.BlockSpec((NC, BM, D_H), lambda i: (0, i, 0)),
            pl.BlockSpec((1, D_H), lambda i: (0, 0)),
            pl.BlockSpec((1, D_H), lambda i: (0, 0)),
            pl.BlockSpec((R, D_H, D_OUT_P), lambda i: (0, 0, 0)),
            pl.BlockSpec((D_H, D_OUT_P), lambda i: (0, 0)),
            pl.BlockSpec((1, D_OUT_P), lambda i: (0, 0)),
        ],
        out_specs=[
            pl.BlockSpec((R, BM, D_OUT_P), lambda i: (0, i, 0)),
            pl.BlockSpec((NC, BM, D_OUT_P), lambda i: (0, i, 0)),
        ],
        out_shape=[
            jax.ShapeDtypeStruct((R, N, D_OUT_P), jnp.float32),
            jax.ShapeDtypeStruct((NC, N, D_OUT_P), jnp.float32),
        ],
    )(acc, gamma, beta, W2p, Ws2p, b2p)


def _finish_body(acc_ref, o_ref):
    s = (acc_ref[0] + acc_ref[1])[:, :D_OUT]
    m = jnp.max(s, axis=1, keepdims=True)
    e = jnp.exp(s - m)
    lse = m + jnp.log(jnp.sum(e, axis=1, keepdims=True))
    o_ref[...] = s - lse


BMF = 2000


@jax.jit
def _finish(acc):
    return pl.pallas_call(
        _finish_body,
        grid=(N // BMF,),
        in_specs=[pl.BlockSpec((NC, BMF, D_OUT_P), lambda i: (0, i, 0))],
        out_specs=pl.BlockSpec((BMF, D_OUT), lambda i: (i, 0)),
        out_shape=jax.ShapeDtypeStruct((N, D_OUT), jnp.float32),
    )(acc)


def _make_agg(D):
    nch = EPT // C
    mesh = plsc.VectorSubcoreMesh(core_axis_name="c", subcore_axis_name="s")

    ETC = 2000
    NB = 3

    @functools.partial(
        pl.kernel,
        out_type=jax.ShapeDtypeStruct((NC, N, D), jnp.float32),
        mesh=mesh,
        scratch_types=[
            pltpu.VMEM((EPT,), jnp.int32),
            pltpu.VMEM((ETC,), jnp.int32),
            [pltpu.VMEM((C,), jnp.int32) for _ in range(NB)],
            [pltpu.VMEM((C, D), jnp.float32) for _ in range(NB)],
            pltpu.VMEM_SHARED((N, D), jnp.float32),
            [pltpu.SemaphoreType.DMA for _ in range(3 * NB)],
        ],
    )
    def agg(ei_hbm, et_hbm, table_hbm, init_hbm, out_hbm,
            gidx_v, et_v, dst_bufs, row_bufs, acc_s, sems):
        c = lax.axis_index("c")
        s = lax.axis_index("s")
        wid = s * NC + c
        ebase = wid * EPT
        sg = sems[0:NB]
        sd = sems[NB:2 * NB]
        ss = sems[2 * NB:]

        pltpu.sync_copy(init_hbm.at[c, pl.ds(s * ROWS_PT, ROWS_PT)],
                        acc_s.at[pl.ds(s * ROWS_PT, ROWS_PT)])

        @pl.when(s == NS - 1)
        def _():
            pltpu.sync_copy(init_hbm.at[c, pl.ds(NS * ROWS_PT, ROWS_TAIL)],
                            acc_s.at[pl.ds(NS * ROWS_PT, ROWS_TAIL)])

        pltpu.sync_copy(ei_hbm.at[pl.ds(ebase, EPT)], gidx_v)

        def stage(j, _):
            pltpu.sync_copy(et_hbm.at[pl.ds(ebase + j * ETC, ETC)], et_v)

            def cvt(i, _):
                sl = pl.ds(j * ETC + i * 16, 16)
                gidx_v[sl] = et_v[pl.ds(i * 16, 16)] * N + gidx_v[sl]
                return 0
            lax.fori_loop(0, ETC // 16, cvt, 0, unroll=8)
            return 0
        lax.fori_loop(0, EPT // ETC, stage, 0)

        plsc.subcore_barrier()

        def gather(k, b):
            return pltpu.make_async_copy(
                table_hbm.at[gidx_v.at[pl.ds(k * C, C)]], row_bufs[b], sg[b])

        def dstcp(k, b):
            return pltpu.make_async_copy(
                ei_hbm.at[pl.ds(E + ebase + k * C, C)], dst_bufs[b], sd[b])

        def scat_start(b):
            pltpu.async_copy(row_bufs[b], acc_s.at[dst_bufs[b]], ss[b],
                             add=True)

        def scat_wait(b):
            pltpu.make_async_copy(row_bufs[b], acc_s.at[dst_bufs[b]],
                                  ss[b]).wait()

        def slot(k, b, bn, first, last):
            if not first:
                scat_wait(bn)
            if not last:
                gather(k + 1, bn).start()
                dstcp(k + 1, bn).start()
            gather(k, b).wait()
            dstcp(k, b).wait()
            scat_start(b)

        gather(0, 0).start()
        dstcp(0, 0).start()
        slot(0, 0, 1, True, False)
        slot(1, 1, 2, True, False)

        def body3(kk, _):
            k = 3 * kk + 2
            slot(k, 2, 0, False, False)
            slot(k + 1, 0, 1, False, False)
            slot(k + 2, 1, 2, False, False)
            return 0
        lax.fori_loop(0, (nch - 5) // 3, body3, 0)

        slot(nch - 3, 2, 0, False, False)
        slot(nch - 2, 0, 1, False, False)
        slot(nch - 1, 1, 2, False, True)
        scat_wait(0)
        scat_wait(1)

        plsc.subcore_barrier()
        pltpu.sync_copy(acc_s.at[pl.ds(s * ROWS_PT, ROWS_PT)],
                        out_hbm.at[c, pl.ds(s * ROWS_PT, ROWS_PT)])

        @pl.when(s == NS - 1)
        def _():
            pltpu.sync_copy(acc_s.at[pl.ds(NS * ROWS_PT, ROWS_TAIL)],
                            out_hbm.at[c, pl.ds(NS * ROWS_PT, ROWS_TAIL)])

    return agg


_agg_h = jax.jit(_make_agg(D_H))
_agg_o = jax.jit(_make_agg(D_OUT_P))


def kernel(x, edge_index1, etypes1, edge_index2, etypes2,
           W1, Ws1, b1, gamma, beta, W2, Ws2, b2):
    T1, init1 = _xform1(x, W1, Ws1, b1.reshape(1, D_H))
    acc1 = _agg_h(edge_index1.reshape(2 * E), etypes1,
                  T1.reshape(R * N, D_H), init1)

    W2p = jnp.pad(W2, ((0, 0), (0, 0), (0, D_OUT_P - D_OUT)))
    Ws2p = jnp.pad(Ws2, ((0, 0), (0, D_OUT_P - D_OUT)))
    b2p = jnp.pad(b2, ((0, D_OUT_P - D_OUT),)).reshape(1, D_OUT_P)
    T2, init2 = _xform2(acc1, gamma.reshape(1, D_H), beta.reshape(1, D_H),
                        W2p, Ws2p, b2p)
    acc2 = _agg_o(edge_index2.reshape(2 * E), etypes2,
                  T2.reshape(R * N, D_OUT_P), init2)
    return _finish(acc2)


# speedup vs baseline: 43.4687x; 1.0742x over previous
import functools

import jax
import jax.numpy as jnp
from jax import lax
from jax.experimental import pallas as pl
from jax.experimental.pallas import tpu as pltpu
from jax.experimental.pallas import tpu_sc as plsc

N = 10000
E = 320000
R = 4
D_IN = 128
D_H = 128
D_OUT = 40
D_OUT_P = 128
BN_EPS = 1e-5

NC = 2
NS = 16
NW = NC * NS
EPT = E // NW
ROWS_PT = 624
ROWS_TAIL = N - NS * ROWS_PT

BM = 2000
C = 80


def _xform1_body(x_ref, w_ref, ws_ref, b_ref, t_ref, s_ref):
    xb = x_ref[...].astype(jnp.bfloat16)
    for r in range(R):
        t_ref[r] = jnp.dot(xb, w_ref[r].astype(jnp.bfloat16),
                           preferred_element_type=jnp.float32)
    s_ref[...] = jnp.dot(xb, ws_ref[...].astype(jnp.bfloat16),
                         preferred_element_type=jnp.float32) + b_ref[...]


@jax.jit
def _xform1(x, W1, Ws1, b1):
    return pl.pallas_call(
        _xform1_body,
        grid=(N // BM,),
        in_specs=[
            pl.BlockSpec((BM, D_IN), lambda i: (i, 0)),
            pl.BlockSpec((R, D_IN, D_H), lambda i: (0, 0, 0)),
            pl.BlockSpec((D_IN, D_H), lambda i: (0, 0)),
            pl.BlockSpec((1, D_H), lambda i: (0, 0)),
        ],
        out_specs=[
            pl.BlockSpec((R, BM, D_H), lambda i: (0, i, 0)),
            pl.BlockSpec((BM, D_H), lambda i: (i, 0)),
        ],
        out_shape=[
            jax.ShapeDtypeStruct((R, N, D_H), jnp.float32),
            jax.ShapeDtypeStruct((N, D_H), jnp.float32),
        ],
    )(x, W1, Ws1, b1)


def _xform2_body(acc_ref, g_ref, be_ref, w_ref, ws_ref, b_ref, t_ref, s_ref):
    inv = 1.0 / jnp.sqrt(1.0 + BN_EPS)
    h = (acc_ref[0] + acc_ref[1]) * inv
    h = g_ref[...] * h + be_ref[...]
    h = jnp.maximum(h, 0.0)
    hb = h.astype(jnp.bfloat16)
    for r in range(R):
        t_ref[r] = jnp.dot(hb, w_ref[r].astype(jnp.bfloat16),
                           preferred_element_type=jnp.float32)
    s_ref[...] = jnp.dot(hb, ws_ref[...].astype(jnp.bfloat16),
                         preferred_element_type=jnp.float32) + b_ref[...]


@jax.jit
def _xform2(acc, gamma, beta, W2p, Ws2p, b2p):
    return pl.pallas_call(
        _xform2_body,
        grid=(N // BM,),
        in_specs=[
            pl.BlockSpec((NC, BM, D_H), lambda i: (0, i, 0)),
            pl.BlockSpec((1, D_H), lambda i: (0, 0)),
            pl.BlockSpec((1, D_H), lambda i: (0, 0)),
            pl.BlockSpec((R, D_H, D_OUT_P), lambda i: (0, 0, 0)),
            pl.BlockSpec((D_H, D_OUT_P), lambda i: (0, 0)),
            pl.BlockSpec((1, D_OUT_P), lambda i: (0, 0)),
        ],
        out_specs=[
            pl.BlockSpec((R, BM, D_OUT_P), lambda i: (0, i, 0)),
            pl.BlockSpec((BM, D_OUT_P), lambda i: (i, 0)),
        ],
        out_shape=[
            jax.ShapeDtypeStruct((R, N, D_OUT_P), jnp.float32),
            jax.ShapeDtypeStruct((N, D_OUT_P), jnp.float32),
        ],
    )(acc, gamma, beta, W2p, Ws2p, b2p)


def _finish_body(acc_ref, o_ref):
    s = (acc_ref[0] + acc_ref[1])[:, :D_OUT]
    m = jnp.max(s, axis=1, keepdims=True)
    e = jnp.exp(s - m)
    lse = m + jnp.log(jnp.sum(e, axis=1, keepdims=True))
    o_ref[...] = s - lse


BMF = 2000


@jax.jit
def _finish(acc):
    return pl.pallas_call(
        _finish_body,
        grid=(N // BMF,),
        in_specs=[pl.BlockSpec((NC, BMF, D_OUT_P), lambda i: (0, i, 0))],
        out_specs=pl.BlockSpec((BMF, D_OUT), lambda i: (i, 0)),
        out_shape=jax.ShapeDtypeStruct((N, D_OUT), jnp.float32),
    )(acc)


def _make_agg(D):
    nch = EPT // C
    mesh = plsc.VectorSubcoreMesh(core_axis_name="c", subcore_axis_name="s")

    ETC = 2000
    NB = 3

    @functools.partial(
        pl.kernel,
        out_type=jax.ShapeDtypeStruct((NC, N, D), jnp.float32),
        mesh=mesh,
        scratch_types=[
            pltpu.VMEM((EPT,), jnp.int32),
            pltpu.VMEM((ETC,), jnp.int32),
            [pltpu.VMEM((C,), jnp.int32) for _ in range(NB)],
            [pltpu.VMEM((C, D), jnp.float32) for _ in range(NB)],
            pltpu.VMEM_SHARED((N, D), jnp.float32),
            [pltpu.SemaphoreType.DMA for _ in range(3 * NB)],
        ],
    )
    def agg(ei_hbm, et_hbm, table_hbm, self_hbm, zero_hbm, out_hbm,
            gidx_v, et_v, dst_bufs, row_bufs, acc_s, sems):
        c = lax.axis_index("c")
        s = lax.axis_index("s")
        wid = s * NC + c
        ebase = wid * EPT
        sg = sems[0:NB]
        sd = sems[NB:2 * NB]
        ss = sems[2 * NB:]

        @pl.when(c == 0)
        def _():
            pltpu.sync_copy(self_hbm.at[pl.ds(s * ROWS_PT, ROWS_PT)],
                            acc_s.at[pl.ds(s * ROWS_PT, ROWS_PT)])

            @pl.when(s == NS - 1)
            def _():
                pltpu.sync_copy(self_hbm.at[pl.ds(NS * ROWS_PT, ROWS_TAIL)],
                                acc_s.at[pl.ds(NS * ROWS_PT, ROWS_TAIL)])

        @pl.when(c == 1)
        def _():
            pltpu.sync_copy(zero_hbm.at[pl.ds(s * ROWS_PT, ROWS_PT)],
                            acc_s.at[pl.ds(s * ROWS_PT, ROWS_PT)])

            @pl.when(s == NS - 1)
            def _():
                pltpu.sync_copy(zero_hbm.at[pl.ds(NS * ROWS_PT, ROWS_TAIL)],
                                acc_s.at[pl.ds(NS * ROWS_PT, ROWS_TAIL)])

        pltpu.sync_copy(ei_hbm.at[pl.ds(ebase, EPT)], gidx_v)

        def stage(j, _):
            pltpu.sync_copy(et_hbm.at[pl.ds(ebase + j * ETC, ETC)], et_v)

            def cvt(i, _):
                sl = pl.ds(j * ETC + i * 16, 16)
                gidx_v[sl] = et_v[pl.ds(i * 16, 16)] * N + gidx_v[sl]
                return 0
            lax.fori_loop(0, ETC // 16, cvt, 0, unroll=8)
            return 0
        lax.fori_loop(0, EPT // ETC, stage, 0)

        plsc.subcore_barrier()

        def gather(k, b):
            return pltpu.make_async_copy(
                table_hbm.at[gidx_v.at[pl.ds(k * C, C)]], row_bufs[b], sg[b])

        def dstcp(k, b):
            return pltpu.make_async_copy(
                ei_hbm.at[pl.ds(E + ebase + k * C, C)], dst_bufs[b], sd[b])

        def scat_start(b):
            pltpu.async_copy(row_bufs[b], acc_s.at[dst_bufs[b]], ss[b],
                             add=True)

        def scat_wait(b):
            pltpu.make_async_copy(row_bufs[b], acc_s.at[dst_bufs[b]],
                                  ss[b]).wait()

        def slot(k, b, bn, first, last):
            if not first:
                scat_wait(bn)
            if not last:
                gather(k + 1, bn).start()
                dstcp(k + 1, bn).start()
            gather(k, b).wait()
            dstcp(k, b).wait()
            scat_start(b)

        gather(0, 0).start()
        dstcp(0, 0).start()
        slot(0, 0, 1, True, False)
        slot(1, 1, 2, True, False)

        def body3(kk, _):
            k = 3 * kk + 2
            slot(k, 2, 0, False, False)
            slot(k + 1, 0, 1, False, False)
            slot(k + 2, 1, 2, False, False)
            return 0
        lax.fori_loop(0, (nch - 5) // 3, body3, 0)

        slot(nch - 3, 2, 0, False, False)
        slot(nch - 2, 0, 1, False, False)
        slot(nch - 1, 1, 2, False, True)
        scat_wait(0)
        scat_wait(1)

        plsc.subcore_barrier()
        pltpu.sync_copy(acc_s.at[pl.ds(s * ROWS_PT, ROWS_PT)],
                        out_hbm.at[c, pl.ds(s * ROWS_PT, ROWS_PT)])

        @pl.when(s == NS - 1)
        def _():
            pltpu.sync_copy(acc_s.at[pl.ds(NS * ROWS_PT, ROWS_TAIL)],
                            out_hbm.at[c, pl.ds(NS * ROWS_PT, ROWS_TAIL)])

    return agg


_agg_h = jax.jit(_make_agg(D_H))
_agg_o = jax.jit(_make_agg(D_OUT_P))


def kernel(x, edge_index1, etypes1, edge_index2, etypes2,
           W1, Ws1, b1, gamma, beta, W2, Ws2, b2):
    zeros = jnp.zeros((N, D_H), jnp.float32)
    T1, self1 = _xform1(x, W1, Ws1, b1.reshape(1, D_H))
    acc1 = _agg_h(edge_index1.reshape(2 * E), etypes1,
                  T1.reshape(R * N, D_H), self1, zeros)

    W2p = jnp.pad(W2, ((0, 0), (0, 0), (0, D_OUT_P - D_OUT)))
    Ws2p = jnp.pad(Ws2, ((0, 0), (0, D_OUT_P - D_OUT)))
    b2p = jnp.pad(b2, ((0, D_OUT_P - D_OUT),)).reshape(1, D_OUT_P)
    T2, self2 = _xform2(acc1, gamma.reshape(1, D_H), beta.reshape(1, D_H),
                        W2p, Ws2p, b2p)
    acc2 = _agg_o(edge_index2.reshape(2 * E), etypes2,
                  T2.reshape(R * N, D_OUT_P), self2, zeros)
    return _finish(acc2)
